# baseline (speedup 1.0000x reference)
import functools

import jax
import jax.numpy as jnp
from jax import lax
from jax.experimental import pallas as pl
from jax.experimental.pallas import tpu as pltpu
from jax.experimental.pallas import tpu_sc as plsc

_N = 4096
_K = 16
_D = 128
_BLK = 256
_BIG = 1e30
_TAKEN = 2e30


def _tc_body(pos_blk_ref, pos_t_ref, pos_ref, sq_col_ref, sq_row_ref,
             bat_col_ref, bat_row_ref, w_ref, b_ref, enc_ref, idx_ref):
    pos_blk = pos_blk_ref[...]
    pos_t = pos_t_ref[...]
    dots = jax.lax.dot_general(pos_blk, pos_t, (((1,), (0,)), ((), ())),
                               preferred_element_type=jnp.float32)
    d2 = sq_col_ref[...] + sq_row_ref[...] - 2.0 * dots
    mask = bat_col_ref[...] != bat_row_ref[...]
    d2 = jnp.where(mask, _BIG, d2)

    col = jax.lax.broadcasted_iota(jnp.int32, (_BLK, _N), 1)
    pos_all = pos_ref[...]
    w = w_ref[...]
    bb = b_ref[...]

    amins = []
    for k in range(_K):
        m = jnp.min(d2, axis=1, keepdims=True)
        cand = jnp.where(d2 == m, col, jnp.int32(_N))
        amin = jnp.min(cand, axis=1, keepdims=True)
        amins.append(amin)
        onehot_b = col == amin
        onehot = onehot_b.astype(jnp.float32)
        d2 = jnp.where(onehot_b, _TAKEN, d2)
        pos_j = jax.lax.dot_general(onehot, pos_all,
                                    (((1,), (0,)), ((), ())),
                                    preferred_element_type=jnp.float32)
        rel = pos_blk - pos_j
        dist = jnp.sqrt(jnp.sum(rel * rel, axis=1, keepdims=True) + 1e-12)
        spatial = jnp.concatenate([pos_blk, pos_j, rel, dist], axis=1)
        enc = jax.lax.dot_general(spatial, w, (((1,), (0,)), ((), ())),
                                  preferred_element_type=jnp.float32)
        enc_ref[:, k, :] = jnp.maximum(enc + bb, 0.0)
    idx_ref[...] = jnp.concatenate(amins, axis=1)


def _tc_stage(x, pos, batch, W, b):
    n = pos.shape[0]
    bat = batch.astype(jnp.int32)
    bat_col = bat.reshape(n, 1)
    bat_row = bat.reshape(1, n)
    pos_t = pos.T
    sq = jnp.sum(pos * pos, axis=-1)
    b2 = b.reshape(1, _D)

    grid = (n // _BLK,)
    enc, idx = pl.pallas_call(
        _tc_body,
        grid=grid,
        in_specs=[
            pl.BlockSpec((_BLK, 3), lambda i: (i, 0)),
            pl.BlockSpec((3, n), lambda i: (0, 0)),
            pl.BlockSpec((n, 3), lambda i: (0, 0)),
            pl.BlockSpec((_BLK, 1), lambda i: (i, 0)),
            pl.BlockSpec((1, n), lambda i: (0, 0)),
            pl.BlockSpec((_BLK, 1), lambda i: (i, 0)),
            pl.BlockSpec((1, n), lambda i: (0, 0)),
            pl.BlockSpec((10, _D), lambda i: (0, 0)),
            pl.BlockSpec((1, _D), lambda i: (0, 0)),
        ],
        out_specs=[
            pl.BlockSpec((_BLK, _K, _D), lambda i: (i, 0, 0)),
            pl.BlockSpec((_BLK, _K), lambda i: (i, 0)),
        ],
        out_shape=[
            jax.ShapeDtypeStruct((n, _K, _D), jnp.float32),
            jax.ShapeDtypeStruct((n, _K), jnp.int32),
        ],
    )(pos, pos_t, pos, sq.reshape(n, 1), sq.reshape(1, n),
      bat_col, bat_row, W, b2)
    return enc, idx


def _sc_gather(x, idx_flat, enc2d):
    nrows = idx_flat.shape[0]
    info = plsc.get_sparse_core_info()
    nc, ns = info.num_cores, info.num_subcores
    nw = nc * ns
    b_per_w = nrows // nw
    sub = 256
    nsub = b_per_w // sub
    mesh = plsc.VectorSubcoreMesh(core_axis_name="c", subcore_axis_name="s")

    @functools.partial(
        pl.kernel, mesh=mesh,
        out_type=jax.ShapeDtypeStruct((nrows, 2 * _D), jnp.float32),
        scratch_types=[
            pltpu.VMEM((b_per_w,), jnp.int32),
            pltpu.VMEM((sub, _D), jnp.float32),
            pltpu.VMEM((sub, _D), jnp.float32),
            pltpu.SemaphoreType.DMA,
        ],
    )
    def g(x_hbm, idx_hbm, enc_hbm, out_hbm, idx_v, xrow_v, enc_v, sem):
        wid = lax.axis_index("s") * nc + lax.axis_index("c")
        base0 = wid * b_per_w
        pltpu.sync_copy(idx_hbm.at[pl.ds(base0, b_per_w)], idx_v)
        for s in range(nsub):
            b0 = base0 + s * sub
            pltpu.async_copy(x_hbm.at[idx_v.at[pl.ds(s * sub, sub)]],
                             xrow_v, sem).wait()
            pltpu.sync_copy(enc_hbm.at[pl.ds(b0, sub)], enc_v)
            pltpu.sync_copy(enc_v, out_hbm.at[pl.ds(b0, sub), pl.ds(0, _D)])
            pltpu.sync_copy(xrow_v, out_hbm.at[pl.ds(b0, sub), pl.ds(_D, _D)])

    return g(x, idx_flat, enc2d)


def kernel(x, pos, batch, W, b):
    n = pos.shape[0]
    enc, idx = _tc_stage(x, pos, batch, W, b)
    out2d = _sc_gather(x, idx.reshape(n * _K), enc.reshape(n * _K, _D))
    return out2d.reshape(n, _K, 2 * _D)


# speedup vs baseline: 1.1024x; 1.1024x over previous
import functools

import jax
import jax.numpy as jnp
from jax import lax
from jax.experimental import pallas as pl
from jax.experimental.pallas import tpu as pltpu
from jax.experimental.pallas import tpu_sc as plsc

_N = 4096
_K = 16
_D = 128
_BLK = 256
_BIG = 1e30
_TAKEN = 2e30


def _tc_body(pos_blk_ref, pos_t_ref, pos_ref, sq_col_ref, sq_row_ref,
             bat_col_ref, bat_row_ref, wac_ref, wbc_ref, w9_ref, b_ref,
             enc_ref, idx_ref):
    pos_blk = pos_blk_ref[...]
    pos_t = pos_t_ref[...]
    dots = jax.lax.dot_general(pos_blk, pos_t, (((1,), (0,)), ((), ())),
                               preferred_element_type=jnp.float32)
    d2 = sq_col_ref[...] + sq_row_ref[...] - 2.0 * dots
    mask = bat_col_ref[...] != bat_row_ref[...]
    d2 = jnp.where(mask, _BIG, d2)

    colf = jax.lax.broadcasted_iota(jnp.int32, (_BLK, _N), 1).astype(jnp.float32)
    pos_all = pos_ref[...]
    base_i = jax.lax.dot_general(pos_blk, wac_ref[...],
                                 (((1,), (0,)), ((), ())),
                                 preferred_element_type=jnp.float32)
    base_i = base_i + b_ref[...]
    wbc = wbc_ref[...]
    w9 = w9_ref[...]

    amins = []
    for k in range(_K):
        m = jnp.min(d2, axis=1, keepdims=True)
        cand = jnp.where(d2 == m, colf, jnp.float32(_N))
        amin = jnp.min(cand, axis=1, keepdims=True)
        amins.append(amin)
        onehot_b = colf == amin
        onehot = onehot_b.astype(jnp.float32)
        d2 = jnp.where(onehot_b, _TAKEN, d2)
        pos_j = jax.lax.dot_general(onehot, pos_all,
                                    (((1,), (0,)), ((), ())),
                                    preferred_element_type=jnp.float32)
        rel = pos_blk - pos_j
        dist = jnp.sqrt(jnp.sum(rel * rel, axis=1, keepdims=True) + 1e-12)
        enc = base_i + jax.lax.dot_general(pos_j, wbc,
                                           (((1,), (0,)), ((), ())),
                                           preferred_element_type=jnp.float32)
        enc_ref[:, k, :] = jnp.maximum(enc + dist * w9, 0.0)
    idx_ref[...] = jnp.concatenate(amins, axis=1).astype(jnp.int32)


def _tc_stage(x, pos, batch, W, b):
    n = pos.shape[0]
    bat = batch.astype(jnp.int32)
    bat_col = bat.reshape(n, 1)
    bat_row = bat.reshape(1, n)
    pos_t = pos.T
    sq = jnp.sum(pos * pos, axis=-1)
    wac = W[0:3] + W[6:9]
    wbc = W[3:6] - W[6:9]
    w9 = W[9:10]
    b2 = b.reshape(1, _D)

    grid = (n // _BLK,)
    enc, idx = pl.pallas_call(
        _tc_body,
        grid=grid,
        in_specs=[
            pl.BlockSpec((_BLK, 3), lambda i: (i, 0)),
            pl.BlockSpec((3, n), lambda i: (0, 0)),
            pl.BlockSpec((n, 3), lambda i: (0, 0)),
            pl.BlockSpec((_BLK, 1), lambda i: (i, 0)),
            pl.BlockSpec((1, n), lambda i: (0, 0)),
            pl.BlockSpec((_BLK, 1), lambda i: (i, 0)),
            pl.BlockSpec((1, n), lambda i: (0, 0)),
            pl.BlockSpec((3, _D), lambda i: (0, 0)),
            pl.BlockSpec((3, _D), lambda i: (0, 0)),
            pl.BlockSpec((1, _D), lambda i: (0, 0)),
            pl.BlockSpec((1, _D), lambda i: (0, 0)),
        ],
        out_specs=[
            pl.BlockSpec((_BLK, _K, _D), lambda i: (i, 0, 0)),
            pl.BlockSpec((_BLK, _K), lambda i: (i, 0)),
        ],
        out_shape=[
            jax.ShapeDtypeStruct((n, _K, _D), jnp.float32),
            jax.ShapeDtypeStruct((n, _K), jnp.int32),
        ],
    )(pos, pos_t, pos, sq.reshape(n, 1), sq.reshape(1, n),
      bat_col, bat_row, wac, wbc, w9, b2)
    return enc, idx


def _sc_gather(x, idx_flat, enc2d):
    nrows = idx_flat.shape[0]
    info = plsc.get_sparse_core_info()
    nc, ns = info.num_cores, info.num_subcores
    nw = nc * ns
    b_per_w = nrows // nw
    sub = 256
    nsub = b_per_w // sub
    mesh = plsc.VectorSubcoreMesh(core_axis_name="c", subcore_axis_name="s")

    @functools.partial(
        pl.kernel, mesh=mesh,
        out_type=jax.ShapeDtypeStruct((nrows, 2 * _D), jnp.float32),
        scratch_types=[
            pltpu.VMEM((b_per_w,), jnp.int32),
            pltpu.VMEM((sub, _D), jnp.float32),
            pltpu.VMEM((sub, _D), jnp.float32),
            pltpu.SemaphoreType.DMA,
        ],
    )
    def g(x_hbm, idx_hbm, enc_hbm, out_hbm, idx_v, xrow_v, enc_v, sem):
        wid = lax.axis_index("s") * nc + lax.axis_index("c")
        base0 = wid * b_per_w
        pltpu.sync_copy(idx_hbm.at[pl.ds(base0, b_per_w)], idx_v)
        for s in range(nsub):
            b0 = base0 + s * sub
            pltpu.async_copy(x_hbm.at[idx_v.at[pl.ds(s * sub, sub)]],
                             xrow_v, sem).wait()
            pltpu.sync_copy(enc_hbm.at[pl.ds(b0, sub)], enc_v)
            pltpu.sync_copy(enc_v, out_hbm.at[pl.ds(b0, sub), pl.ds(0, _D)])
            pltpu.sync_copy(xrow_v, out_hbm.at[pl.ds(b0, sub), pl.ds(_D, _D)])

    return g(x, idx_flat, enc2d)


def kernel(x, pos, batch, W, b):
    n = pos.shape[0]
    enc, idx = _tc_stage(x, pos, batch, W, b)
    out2d = _sc_gather(x, idx.reshape(n * _K), enc.reshape(n * _K, _D))
    return out2d.reshape(n, _K, 2 * _D)


# speedup vs baseline: 1.2047x; 1.0928x over previous
import jax
import jax.numpy as jnp
from jax.experimental import pallas as pl

_N = 4096
_K = 16
_D = 128
_BLK = 256
_BIG = 1e30
_TAKEN = 2e30


def _body(pos_blk_ref, pos_t_ref, pos_ref, sq_col_ref, sq_row_ref,
          bat_col_ref, bat_row_ref, x_ref, wac_ref, wbc_ref, w9_ref, b_ref,
          out_ref):
    pos_blk = pos_blk_ref[...]
    pos_t = pos_t_ref[...]
    dots = jax.lax.dot_general(pos_blk, pos_t, (((1,), (0,)), ((), ())),
                               preferred_element_type=jnp.float32)
    d2 = sq_col_ref[...] + sq_row_ref[...] - 2.0 * dots
    mask = bat_col_ref[...] != bat_row_ref[...]
    d2 = jnp.where(mask, _BIG, d2)

    colf = jax.lax.broadcasted_iota(jnp.int32, (_BLK, _N), 1).astype(jnp.float32)
    pos_all = pos_ref[...]
    x_all = x_ref[...]
    base_i = jax.lax.dot_general(pos_blk, wac_ref[...],
                                 (((1,), (0,)), ((), ())),
                                 preferred_element_type=jnp.float32)
    base_i = base_i + b_ref[...]
    wbc = wbc_ref[...]
    w9 = w9_ref[...]

    for k in range(_K):
        m = jnp.min(d2, axis=1, keepdims=True)
        cand = jnp.where(d2 == m, colf, jnp.float32(_N))
        amin = jnp.min(cand, axis=1, keepdims=True)
        onehot_b = colf == amin
        onehot = onehot_b.astype(jnp.float32)
        d2 = jnp.where(onehot_b, _TAKEN, d2)
        pos_j = jax.lax.dot_general(onehot, pos_all,
                                    (((1,), (0,)), ((), ())),
                                    preferred_element_type=jnp.float32)
        x_j = jax.lax.dot_general(onehot, x_all,
                                  (((1,), (0,)), ((), ())),
                                  preferred_element_type=jnp.float32)
        rel = pos_blk - pos_j
        dist = jnp.sqrt(jnp.sum(rel * rel, axis=1, keepdims=True) + 1e-12)
        enc = base_i + jax.lax.dot_general(pos_j, wbc,
                                           (((1,), (0,)), ((), ())),
                                           preferred_element_type=jnp.float32)
        out_ref[:, k, 0:_D] = jnp.maximum(enc + dist * w9, 0.0)
        out_ref[:, k, _D:2 * _D] = x_j


def kernel(x, pos, batch, W, b):
    n = pos.shape[0]
    bat = batch.astype(jnp.int32)
    bat_col = bat.reshape(n, 1)
    bat_row = bat.reshape(1, n)
    pos_t = pos.T
    sq = jnp.sum(pos * pos, axis=-1)
    wac = W[0:3] + W[6:9]
    wbc = W[3:6] - W[6:9]
    w9 = W[9:10]
    b2 = b.reshape(1, _D)

    grid = (n // _BLK,)
    out = pl.pallas_call(
        _body,
        grid=grid,
        in_specs=[
            pl.BlockSpec((_BLK, 3), lambda i: (i, 0)),
            pl.BlockSpec((3, n), lambda i: (0, 0)),
            pl.BlockSpec((n, 3), lambda i: (0, 0)),
            pl.BlockSpec((_BLK, 1), lambda i: (i, 0)),
            pl.BlockSpec((1, n), lambda i: (0, 0)),
            pl.BlockSpec((_BLK, 1), lambda i: (i, 0)),
            pl.BlockSpec((1, n), lambda i: (0, 0)),
            pl.BlockSpec((n, _D), lambda i: (0, 0)),
            pl.BlockSpec((3, _D), lambda i: (0, 0)),
            pl.BlockSpec((3, _D), lambda i: (0, 0)),
            pl.BlockSpec((1, _D), lambda i: (0, 0)),
            pl.BlockSpec((1, _D), lambda i: (0, 0)),
        ],
        out_specs=pl.BlockSpec((_BLK, _K, 2 * _D), lambda i: (i, 0, 0)),
        out_shape=jax.ShapeDtypeStruct((n, _K, 2 * _D), jnp.float32),
    )(pos, pos_t, pos, sq.reshape(n, 1), sq.reshape(1, n),
      bat_col, bat_row, x, wac, wbc, w9, b2)
    return out
